# baseline (speedup 1.0000x reference)
import functools

import jax
import jax.numpy as jnp
from jax import lax
from jax.experimental import pallas as pl
from jax.experimental.pallas import tpu as pltpu
from jax.experimental.pallas import tpu_sc as plsc

ROWS_IN = 500000
ROWS_OUT = 250000
D = 128
L = 16
C = 200
NCHUNK = ROWS_OUT // C
TAIL = ROWS_OUT - NCHUNK * C
TAIL_BASE = NCHUNK * C
NC = 2
NS = 16
NW = NC * NS
TAIL_WID = NCHUNK % NW


def _sc_body(x_hbm, out_hbm, idx0, idx1, rows0, rows1, gsem0, gsem1, wsem0, wsem1):
    wid = lax.axis_index("s") * NC + lax.axis_index("c")
    niter = (NCHUNK - wid + NW - 1) // NW

    lane2 = 2 * lax.iota(jnp.int32, L)

    def build_idx(idx_v, base, n):
        base2 = 2 * base
        for j in range(n // L):
            idx_v[pl.ds(j * L, L)] = base2 + 2 * j * L + lane2
        if n % L:
            off = n - L
            idx_v[pl.ds(off, L)] = base2 + 2 * off + lane2

    def start_gather(c, idx_v, rows_v, gsem):
        build_idx(idx_v, c * C, C)
        pltpu.async_copy(x_hbm.at[idx_v], rows_v, gsem)

    bufs = ((idx0, rows0, gsem0, wsem0), (idx1, rows1, gsem1, wsem1))

    start_gather(wid, idx0, rows0, gsem0)

    def chunk_body(k, _):
        def step(p):
            idx_v, rows_v, gsem, wsem = bufs[p]
            o_idx, o_rows, o_gsem, o_wsem = bufs[1 - p]

            pltpu.make_async_copy(x_hbm.at[idx_v], rows_v, gsem).wait()
            c = wid + k * NW
            pltpu.async_copy(rows_v, out_hbm.at[pl.ds(c * C, C)], wsem)

            @pl.when(k + 1 < niter)
            def _():
                @pl.when(k >= 1)
                def _():
                    pltpu.make_async_copy(
                        o_rows, out_hbm.at[pl.ds(0, C)], o_wsem
                    ).wait()

                start_gather(wid + (k + 1) * NW, o_idx, o_rows, o_gsem)

        @pl.when(k % 2 == 0)
        def _():
            step(0)

        @pl.when(k % 2 == 1)
        def _():
            step(1)

        return 0

    lax.fori_loop(0, niter, chunk_body, 0)
    pltpu.make_async_copy(rows0, out_hbm.at[pl.ds(0, C)], wsem0).wait()
    pltpu.make_async_copy(rows1, out_hbm.at[pl.ds(0, C)], wsem1).wait()

    if TAIL:
        @pl.when(wid == TAIL_WID)
        def _():
            build_idx(idx0, TAIL_BASE, TAIL)
            tail_rows = rows0.at[pl.ds(0, TAIL)]
            pltpu.async_copy(
                x_hbm.at[idx0.at[pl.ds(0, TAIL)]], tail_rows, gsem0
            ).wait()
            pltpu.sync_copy(tail_rows, out_hbm.at[pl.ds(TAIL_BASE, TAIL)])


def kernel(x):
    mesh = plsc.VectorSubcoreMesh(core_axis_name="c", subcore_axis_name="s")
    run = pl.kernel(
        _sc_body,
        mesh=mesh,
        out_type=jax.ShapeDtypeStruct((ROWS_OUT, D), jnp.float32),
        scratch_types=[
            pltpu.VMEM((C,), jnp.int32),
            pltpu.VMEM((C,), jnp.int32),
            pltpu.VMEM((C, D), jnp.float32),
            pltpu.VMEM((C, D), jnp.float32),
            pltpu.SemaphoreType.DMA,
            pltpu.SemaphoreType.DMA,
            pltpu.SemaphoreType.DMA,
            pltpu.SemaphoreType.DMA,
        ],
    )
    return run(x)


# speedup vs baseline: 1.0758x; 1.0758x over previous
import functools

import jax
import jax.numpy as jnp
from jax import lax
from jax.experimental import pallas as pl
from jax.experimental.pallas import tpu as pltpu
from jax.experimental.pallas import tpu_sc as plsc

ROWS_IN = 500000
ROWS_OUT = 250000
D = 128
L = 16
C = 400
NCHUNK = ROWS_OUT // C
TAIL = ROWS_OUT - NCHUNK * C
TAIL_BASE = NCHUNK * C
NC = 2
NS = 16
NW = NC * NS
TAIL_WID = NCHUNK % NW


def _sc_body(x_hbm, out_hbm, idx0, idx1, rows0, rows1, gsem0, gsem1, wsem0, wsem1):
    wid = lax.axis_index("s") * NC + lax.axis_index("c")
    niter = (NCHUNK - wid + NW - 1) // NW

    lane2 = 2 * lax.iota(jnp.int32, L)

    def build_idx(idx_v, base, n):
        base2 = 2 * base
        for j in range(n // L):
            idx_v[pl.ds(j * L, L)] = base2 + 2 * j * L + lane2
        if n % L:
            off = n - L
            idx_v[pl.ds(off, L)] = base2 + 2 * off + lane2

    def start_gather(c, idx_v, rows_v, gsem):
        build_idx(idx_v, c * C, C)
        pltpu.async_copy(x_hbm.at[idx_v], rows_v, gsem)

    bufs = ((idx0, rows0, gsem0, wsem0), (idx1, rows1, gsem1, wsem1))

    start_gather(wid, idx0, rows0, gsem0)

    def chunk_body(k, _):
        def step(p):
            idx_v, rows_v, gsem, wsem = bufs[p]
            o_idx, o_rows, o_gsem, o_wsem = bufs[1 - p]

            @pl.when(k + 1 < niter)
            def _():
                @pl.when(k >= 1)
                def _():
                    pltpu.make_async_copy(
                        o_rows, out_hbm.at[pl.ds(0, C)], o_wsem
                    ).wait()

                start_gather(wid + (k + 1) * NW, o_idx, o_rows, o_gsem)

            pltpu.make_async_copy(x_hbm.at[idx_v], rows_v, gsem).wait()
            c = wid + k * NW
            pltpu.async_copy(rows_v, out_hbm.at[pl.ds(c * C, C)], wsem)

        @pl.when(k % 2 == 0)
        def _():
            step(0)

        @pl.when(k % 2 == 1)
        def _():
            step(1)

        return 0

    lax.fori_loop(0, niter, chunk_body, 0)
    pltpu.make_async_copy(rows0, out_hbm.at[pl.ds(0, C)], wsem0).wait()
    pltpu.make_async_copy(rows1, out_hbm.at[pl.ds(0, C)], wsem1).wait()

    if TAIL:
        @pl.when(wid == TAIL_WID)
        def _():
            build_idx(idx0, TAIL_BASE, TAIL)
            tail_rows = rows0.at[pl.ds(0, TAIL)]
            pltpu.async_copy(
                x_hbm.at[idx0.at[pl.ds(0, TAIL)]], tail_rows, gsem0
            ).wait()
            pltpu.sync_copy(tail_rows, out_hbm.at[pl.ds(TAIL_BASE, TAIL)])


def kernel(x):
    mesh = plsc.VectorSubcoreMesh(core_axis_name="c", subcore_axis_name="s")
    run = pl.kernel(
        _sc_body,
        mesh=mesh,
        out_type=jax.ShapeDtypeStruct((ROWS_OUT, D), jnp.float32),
        scratch_types=[
            pltpu.VMEM((C,), jnp.int32),
            pltpu.VMEM((C,), jnp.int32),
            pltpu.VMEM((C, D), jnp.float32),
            pltpu.VMEM((C, D), jnp.float32),
            pltpu.SemaphoreType.DMA,
            pltpu.SemaphoreType.DMA,
            pltpu.SemaphoreType.DMA,
            pltpu.SemaphoreType.DMA,
        ],
    )
    return run(x)


# speedup vs baseline: 1.0849x; 1.0085x over previous
import functools

import jax
import jax.numpy as jnp
from jax import lax
from jax.experimental import pallas as pl
from jax.experimental.pallas import tpu as pltpu
from jax.experimental.pallas import tpu_sc as plsc

ROWS_IN = 500000
ROWS_OUT = 250000
D = 128
L = 16
NBUF = 4
C = 240
NCHUNK = ROWS_OUT // C
TAIL = ROWS_OUT - NCHUNK * C
TAIL_BASE = NCHUNK * C
NC = 2
NS = 16
NW = NC * NS
TAIL_WID = NCHUNK % NW


def _sc_body(x_hbm, out_hbm, *scratch):
    idxs = scratch[0:NBUF]
    rows = scratch[NBUF:2 * NBUF]
    gsems = scratch[2 * NBUF:3 * NBUF]
    wsems = scratch[3 * NBUF:4 * NBUF]

    wid = lax.axis_index("s") * NC + lax.axis_index("c")
    niter = (NCHUNK - wid + NW - 1) // NW

    lane2 = 2 * lax.iota(jnp.int32, L)

    def build_idx(idx_v, base, n):
        base2 = 2 * base
        for j in range(n // L):
            idx_v[pl.ds(j * L, L)] = base2 + 2 * j * L + lane2
        if n % L:
            off = n - L
            idx_v[pl.ds(off, L)] = base2 + 2 * off + lane2

    def start_gather(c, idx_v, rows_v, gsem):
        build_idx(idx_v, c * C, C)
        pltpu.async_copy(x_hbm.at[idx_v], rows_v, gsem)

    start_gather(wid, idxs[0], rows[0], gsems[0])

    def chunk_body(k, _):
        def step(p):
            nxt = (p + 1) % NBUF

            @pl.when(k + 1 < niter)
            def _():
                @pl.when(k + 1 >= NBUF)
                def _():
                    pltpu.make_async_copy(
                        rows[nxt], out_hbm.at[pl.ds(0, C)], wsems[nxt]
                    ).wait()

                start_gather(wid + (k + 1) * NW, idxs[nxt], rows[nxt], gsems[nxt])

            pltpu.make_async_copy(x_hbm.at[idxs[p]], rows[p], gsems[p]).wait()
            c = wid + k * NW
            pltpu.async_copy(rows[p], out_hbm.at[pl.ds(c * C, C)], wsems[p])

        for p in range(NBUF):
            @pl.when(k % NBUF == p)
            def _(p=p):
                step(p)

        return 0

    lax.fori_loop(0, niter, chunk_body, 0)
    for p in range(NBUF):
        pltpu.make_async_copy(rows[p], out_hbm.at[pl.ds(0, C)], wsems[p]).wait()

    if TAIL:
        @pl.when(wid == TAIL_WID)
        def _():
            build_idx(idxs[0], TAIL_BASE, TAIL)
            tail_rows = rows[0].at[pl.ds(0, TAIL)]
            pltpu.async_copy(
                x_hbm.at[idxs[0].at[pl.ds(0, TAIL)]], tail_rows, gsems[0]
            ).wait()
            pltpu.sync_copy(tail_rows, out_hbm.at[pl.ds(TAIL_BASE, TAIL)])


def kernel(x):
    mesh = plsc.VectorSubcoreMesh(core_axis_name="c", subcore_axis_name="s")
    run = pl.kernel(
        _sc_body,
        mesh=mesh,
        out_type=jax.ShapeDtypeStruct((ROWS_OUT, D), jnp.float32),
        scratch_types=(
            [pltpu.VMEM((C,), jnp.int32) for _ in range(NBUF)]
            + [pltpu.VMEM((C, D), jnp.float32) for _ in range(NBUF)]
            + [pltpu.SemaphoreType.DMA for _ in range(2 * NBUF)]
        ),
    )
    return run(x)
